# baseline (speedup 1.0000x reference)
import functools

import jax
import jax.numpy as jnp
from jax import lax
from jax.experimental import pallas as pl
from jax.experimental.pallas import tpu as pltpu
from jax.experimental.pallas import tpu_sc as plsc

F = 128
H = 64
RNG = 12544
NPS, ERS, NRS = 50176, 802816, 392
NPU, ERU, NRU = 100352, 1601536, 784
_B, _BPW = 4096, 128
CH = 64

_CP = pltpu.CompilerParams(use_tc_tiling_on_sc=False, needs_layout_passes=False)


def _mesh():
    return plsc.VectorSubcoreMesh(core_axis_name="c", subcore_axis_name="s")


def _wid():
    return lax.axis_index("s") * 2 + lax.axis_index("c")


def _lane(v, j):
    jj = jnp.full((16, 1), j, jnp.int32)
    return lax.gather(
        v, jj,
        lax.GatherDimensionNumbers(offset_dims=(), collapsed_slice_dims=(0,),
                                   start_index_map=(0,)),
        (1,), mode=lax.GatherScatterMode.PROMISE_IN_BOUNDS)


def _z16():
    return jnp.zeros((16,), jnp.float32)


def _ones16():
    return jnp.ones((16,), jnp.float32)


def _deg(n_rows, e_pad, two):
    ept = e_pad // 32
    nch = ept // 128
    nh = 2 if two else 1
    out_type = tuple(jax.ShapeDtypeStruct((32, n_rows, F), jnp.float32)
                     for _ in range(nh))
    scratch = ([pltpu.VMEM((n_rows, F), jnp.float32)] * nh
               + [pltpu.VMEM((128,), jnp.int32)] * nh)

    @functools.partial(pl.kernel, out_type=out_type, mesh=_mesh(),
                       scratch_types=scratch, compiler_params=_CP)
    def k(*refs):
        ins = refs[:nh]
        outs = refs[nh:2 * nh]
        hists = refs[2 * nh:3 * nh]
        idxs = refs[3 * nh:4 * nh]
        w = _wid()

        def zb(i, _):
            for hh in hists:
                for q in range(8):
                    hh[i, pl.ds(q * 16, 16)] = _z16()
            return _
        lax.fori_loop(0, n_rows, zb, None)

        def eb(ch, _):
            off = w * ept + ch * 128
            for t in range(nh):
                pltpu.sync_copy(ins[t].at[pl.ds(off, 128)], idxs[t])
            for t in range(nh):
                for j in range(8):
                    v = idxs[t][pl.ds(j * 16, 16)]
                    plsc.addupdate_scatter(
                        hists[t],
                        [jnp.right_shift(v, 7), jnp.bitwise_and(v, 127)],
                        _ones16())
            return _
        lax.fori_loop(0, nch, eb, None)
        for t in range(nh):
            pltpu.sync_copy(hists[t], outs[t].at[w])

    return k


def _rsqrt(n_rows, nh):
    rb = n_rows // 7

    def body(*refs):
        for t in range(nh):
            deg = jnp.sum(refs[t][...], axis=0)
            refs[nh + t][...] = lax.rsqrt(jnp.maximum(deg, 1.0))

    return pl.pallas_call(
        body, grid=(7,),
        in_specs=[pl.BlockSpec((32, rb, F), lambda i: (0, i, 0))] * nh,
        out_specs=[pl.BlockSpec((rb, F), lambda i: (i, 0))] * nh,
        out_shape=[jax.ShapeDtypeStruct((n_rows, F), jnp.float32)] * nh)


def _scale(n_pad):
    npt = n_pad // 32
    nch = npt // 16
    scratch = [pltpu.VMEM((16, F), jnp.float32), pltpu.VMEM((16,), jnp.float32)]

    @functools.partial(
        pl.kernel, out_type=jax.ShapeDtypeStruct((n_pad, F), jnp.float32),
        mesh=_mesh(), scratch_types=scratch, compiler_params=_CP)
    def k(x_hbm, s_hbm, g_hbm, xr, sv):
        base = _wid() * npt

        def body(ch, _):
            r0 = base + ch * 16
            pltpu.sync_copy(x_hbm.at[pl.ds(r0, 16)], xr)
            pltpu.sync_copy(s_hbm.at[pl.ds(r0, 16)], sv)
            sval = sv[...]
            for j in range(16):
                b = _lane(sval, j)
                for q in range(8):
                    xr[j, pl.ds(q * 16, 16)] = xr[j, pl.ds(q * 16, 16)] * b
            pltpu.sync_copy(xr, g_hbm.at[pl.ds(r0, 16)])
            return _
        lax.fori_loop(0, nch, body, None)

    return k


def _bin(n_pad, e_pad, nb):
    ept2 = e_pad // 32
    nch = ept2 // CH
    cap = ept2 + 128
    i32 = jnp.int32
    out_type = (jax.ShapeDtypeStruct((nb, 32, cap), i32),
                jax.ShapeDtypeStruct((32, 16), i32))
    scratch = [
        pltpu.VMEM((nb, 160), i32),
        pltpu.VMEM((CH,), i32),
        pltpu.VMEM((CH,), i32),
        pltpu.VMEM((16,), i32),
    ]

    @functools.partial(pl.kernel, out_type=out_type, mesh=_mesh(),
                       scratch_types=scratch, compiler_params=_CP)
    def k(srcp, dstp, bp, lens, sbuf, srcv, dstv, lenv):
        w = _wid()
        tb = w * ept2

        def body(ch, carry):
            ptrs = list(carry[:nb])
            wcs = list(carry[nb:])
            off = tb + ch * CH
            pltpu.sync_copy(srcp.at[pl.ds(off, CH)], srcv)
            pltpu.sync_copy(dstp.at[pl.ds(off, CH)], dstv)
            for g in range(CH // 16):
                s16 = srcv[pl.ds(g * 16, 16)]
                d16 = dstv[pl.ds(g * 16, 16)]
                for b in range(nb):
                    lo = b * RNG
                    m = jnp.logical_and(d16 >= lo, d16 < lo + RNG)
                    pk = jnp.bitwise_or(
                        s16, jnp.left_shift(d16 - lo, 17))
                    plsc.store_compressed(sbuf.at[b, pl.ds(ptrs[b], 16)],
                                          pk, mask=m)
                    cnt = lax.reduce_max(
                        plsc.all_reduce_population_count(m), (0,))
                    ptr = ptrs[b] + cnt
                    full = ptr >= 128

                    @pl.when(full)
                    def _(b=b, wc=wcs[b]):
                        pltpu.sync_copy(sbuf.at[b, pl.ds(0, 128)],
                                        bp.at[b, w, pl.ds(wc * 128, 128)])
                        ts = sbuf[b, pl.ds(128, 16)]
                        sbuf[b, pl.ds(0, 16)] = ts
                    ptrs[b] = jnp.where(full, ptr - 128, ptr)
                    wcs[b] = wcs[b] + full.astype(i32)
            return tuple(ptrs) + tuple(wcs)

        z = jnp.zeros((), i32)
        carry = lax.fori_loop(0, nch, body, (z,) * (2 * nb))

        lvec = jnp.zeros((16,), i32)
        for b in range(nb):
            ptr = carry[b]
            wc = carry[nb + b]
            dm16 = jnp.full((16,), (n_pad - 1) | (RNG << 17), i32)
            for kq in range(8):
                @pl.when(ptr + kq * 16 < 128)
                def _(b=b, o=ptr + kq * 16):
                    sbuf[b, pl.ds(o, 16)] = dm16

            @pl.when(ptr > 0)
            def _(b=b, wc=wc):
                pltpu.sync_copy(sbuf.at[b, pl.ds(0, 128)],
                                bp.at[b, w, pl.ds(wc * 128, 128)])
            nchunks = wc + (ptr > 0).astype(i32)
            lvec = lvec + nchunks * jnp.where(
                lax.iota(i32, 16) == b, 1, 0)
        lenv[...] = lvec
        pltpu.sync_copy(lenv, lens.at[w])

    return k


def _hop(n_pad, e_pad, emit_g):
    nb = n_pad // RNG
    npp = nb // 2
    cap = e_pad // 32 + 128
    rpt = RNG // 16
    nfl = rpt // 16
    HC = 32
    out_type = tuple(jax.ShapeDtypeStruct((n_pad, F), jnp.float32)
                     for _ in range(2 if emit_g else 1))
    scratch = (
        [pltpu.VMEM_SHARED((RNG + 8, F), jnp.float32)]
        + [pltpu.VMEM((HC,), jnp.int32)] * 4
        + [pltpu.VMEM((HC,), jnp.int32)]
        + [pltpu.VMEM((HC,), jnp.int32)] * 4
        + [pltpu.VMEM((HC, F), jnp.float32)] * 4
        + [
            pltpu.VMEM((16, F), jnp.float32),
            pltpu.VMEM((16, F), jnp.float32),
            pltpu.VMEM((16, F), jnp.float32),
            pltpu.VMEM((16, F), jnp.float32),
            pltpu.VMEM((16,), jnp.float32),
            pltpu.VMEM((16,), jnp.float32),
            pltpu.VMEM((16,), jnp.int32),
            pltpu.VMEM((16,), jnp.int32),
        ]
        + [pltpu.SemaphoreType.DMA] * 4
    )

    @functools.partial(pl.kernel, out_type=out_type, mesh=_mesh(),
                       scratch_types=scratch, compiler_params=_CP)
    def k(*refs):
        g_hbm, bp_hbm, lens_hbm, si_hbm, so_hbm = refs[:5]
        if emit_g:
            h_hbm, gn_hbm = refs[5:7]
            sc = refs[7:]
        else:
            h_hbm = refs[5]
            gn_hbm = None
            sc = refs[6:]
        (acc, srcv0, srcv1, srcv2, srcv3, dstv, dstl0, dstl1, dstl2, dstl3,
         rows0, rows1, rows2, rows3, tv, hv, gv, z16, siv, sov,
         lenv0, lenv1, sem0, sem1, sem2, sem3) = sc
        bufs = ((srcv0, dstl0, rows0, sem0), (srcv1, dstl1, rows1, sem1),
                (srcv2, dstl2, rows2, sem2), (srcv3, dstl3, rows3, sem3))
        cid = lax.axis_index("c")
        tid = lax.axis_index("s")

        for r in range(16):
            for q in range(8):
                z16[r, pl.ds(q * 16, 16)] = _z16()

        def zinit(q, _):
            pltpu.sync_copy(z16, acc.at[pl.ds(tid * rpt + q * 16, 16)])
            return _
        lax.fori_loop(0, nfl, zinit, None)

        @pl.when(tid == 0)
        def _():
            pltpu.sync_copy(z16.at[pl.ds(0, 8)], acc.at[pl.ds(RNG, 8)])
        plsc.subcore_barrier()

        i32 = jnp.int32
        pltpu.sync_copy(lens_hbm.at[2 * tid], lenv0)
        pltpu.sync_copy(lens_hbm.at[2 * tid + 1], lenv1)
        lv0 = lenv0[...]
        lv1 = lenv1[...]

        for p in range(npp):
            bkt = cid * npp + p
            base = bkt * RNG
            oh0 = jnp.where(lax.iota(i32, 16) == p, 1, 0)
            oh1 = jnp.where(lax.iota(i32, 16) == npp + p, 1, 0)
            oh = jnp.where(cid == 0, oh0, oh1)
            n0 = lax.reduce_max(lv0 * oh, (0,)) * (128 // HC)
            n1 = lax.reduce_max(lv1 * oh, (0,)) * (128 // HC)
            ntot = n0 + n1

            def prep(c, k):
                srcv, dstl, rows, sem = bufs[k]
                in1 = (c >= n0).astype(i32)
                w = 2 * tid + in1
                o = (c - in1 * n0) * HC
                pltpu.sync_copy(bp_hbm.at[bkt, w, pl.ds(o, HC)], dstv)
                for j in range(HC // 16):
                    pk = dstv[pl.ds(j * 16, 16)]
                    srcv[pl.ds(j * 16, 16)] = jnp.bitwise_and(pk, 0x1FFFF)
                    dstl[pl.ds(j * 16, 16)] = jnp.right_shift(pk, 17)
                return pltpu.async_copy(g_hbm.at[srcv], rows, sem)

            def drain(k):
                srcv, dstl, rows, sem = bufs[k]
                pltpu.make_async_copy(g_hbm.at[srcv], rows, sem).wait()
                pltpu.sync_copy(rows, acc.at[dstl], add=True)

            for k in range(3):
                @pl.when(k < ntot)
                def _(k=k):
                    prep(jnp.full((), k, i32), k)

            def scan_body(i4, _):
                c = 4 * i4

                @pl.when(c + 3 < ntot)
                def _():
                    prep(c + 3, 3)
                drain(0)

                @pl.when(c + 4 < ntot)
                def _():
                    prep(c + 4, 0)

                @pl.when(c + 1 < ntot)
                def _():
                    drain(1)

                @pl.when(c + 5 < ntot)
                def _():
                    prep(c + 5, 1)

                @pl.when(c + 2 < ntot)
                def _():
                    drain(2)

                @pl.when(c + 6 < ntot)
                def _():
                    prep(c + 6, 2)

                @pl.when(c + 3 < ntot)
                def _():
                    drain(3)
                return _
            lax.fori_loop(0, (ntot + 3) // 4, scan_body, None)
            plsc.subcore_barrier()

            def fl(q, _):
                r0 = tid * rpt + q * 16
                n0 = base + r0
                pltpu.sync_copy(acc.at[pl.ds(r0, 16)], tv)
                pltpu.sync_copy(z16, acc.at[pl.ds(r0, 16)])
                pltpu.sync_copy(si_hbm.at[pl.ds(n0, 16)], siv)
                if emit_g:
                    pltpu.sync_copy(so_hbm.at[pl.ds(n0, 16)], sov)
                sival = siv[...]
                soval = sov[...] if emit_g else None
                for j in range(16):
                    a = _lane(sival, j)
                    if emit_g:
                        b = _lane(soval, j) * a
                    for q2 in range(8):
                        t = tv[j, pl.ds(q2 * 16, 16)]
                        hv[j, pl.ds(q2 * 16, 16)] = t * a
                        if emit_g:
                            gv[j, pl.ds(q2 * 16, 16)] = t * b
                pltpu.sync_copy(hv, h_hbm.at[pl.ds(n0, 16)])
                if emit_g:
                    pltpu.sync_copy(gv, gn_hbm.at[pl.ds(n0, 16)])
                return _
            lax.fori_loop(0, nfl, fl, None)

            @pl.when(tid == 0)
            def _():
                pltpu.sync_copy(z16.at[pl.ds(0, 8)], acc.at[pl.ds(RNG, 8)])
            plsc.subcore_barrier()

    return k


def _assemble(xs, h1s, h2s, xu, h1u, h2u):
    rb = 2000
    grid = 50000 // rb

    def body(axs, ah1s, ah2s, au0, au1, au2, ai0, ai1, ai2,
             s64, soc, u64, a64, itm, ru, ri):
        fs = (axs[...] + ah1s[...] + ah2s[...]) * (1.0 / 3.0)
        fu = (au0[...] + au1[...] + au2[...]) * (1.0 / 3.0)
        fi = (ai0[...] + ai1[...] + ai2[...]) * (1.0 / 3.0)
        s64[...] = fs[:, :H]
        soc[...] = fs[:, H:]
        a64[...] = fu[:, :H]
        ru[...] = fu[:, H:]
        itm[...] = fi[:, :H]
        ri[...] = fi[:, H:]
        u64[...] = 0.5 * fs[:, :H] + 0.5 * fu[:, :H]

    uspec = pl.BlockSpec((rb, F), lambda i: (i, 0))
    ispec = pl.BlockSpec((rb, F), lambda i: (grid + i, 0))
    ospec = pl.BlockSpec((rb, H), lambda i: (i, 0))
    return pl.pallas_call(
        body, grid=(grid,),
        in_specs=[uspec] * 6 + [ispec] * 3,
        out_specs=[ospec] * 7,
        out_shape=[jax.ShapeDtypeStruct((50000, H), jnp.float32)] * 7,
        compiler_params=pltpu.CompilerParams(
            vmem_limit_bytes=100 * 1024 * 1024),
    )(xs, h1s, h2s, xu, h1u, h2u, xu, h1u, h2u)


def _gather_many(tables, idxs):
    nt = len(tables)
    out_type = tuple(
        jax.ShapeDtypeStruct((idxs[t].shape[0], tables[t].shape[1]),
                             jnp.float32) for t in range(nt))
    scratch = [
        pltpu.VMEM((_BPW,), jnp.int32),
        pltpu.VMEM((_BPW, H), jnp.float32),
        pltpu.SemaphoreType.DMA,
    ]

    @functools.partial(pl.kernel, out_type=out_type, mesh=_mesh(),
                       scratch_types=scratch, compiler_params=_CP)
    def k(*refs):
        tabs = refs[:nt]
        idxr = refs[nt:2 * nt]
        outs = refs[2 * nt:3 * nt]
        idx_v, rows_v, sem = refs[3 * nt:]
        base = _wid() * _BPW
        for t in range(nt):
            pltpu.sync_copy(idxr[t].at[pl.ds(base, _BPW)], idx_v)
            pltpu.async_copy(tabs[t].at[idx_v], rows_v, sem).wait()
            pltpu.sync_copy(rows_v, outs[t].at[pl.ds(base, _BPW)])

    return k(*tables, *idxs)


def kernel(users, pos, neg, social_edge_index, ui_edge_index,
           user_w, item_w, user1_w, item1_w, user2_w, item2_w):
    i32 = jnp.int32
    f32 = jnp.float32
    users = users.astype(i32)
    pos = pos.astype(i32)
    neg = neg.astype(i32)

    s_src = social_edge_index[0].astype(i32)
    s_dst = social_edge_index[1].astype(i32)
    pad_s = jnp.full((ERS - s_src.shape[0],), NPS - 1, i32)
    s_srcp = jnp.concatenate([s_src, pad_s])
    s_dstp = jnp.concatenate([s_dst, pad_s])

    uu = ui_edge_index[0].astype(i32)
    ii = ui_edge_index[1].astype(i32) + 50000
    pad_u = jnp.full((ERU - 2 * uu.shape[0],), NPU - 1, i32)
    b_srcp = jnp.concatenate([uu, ii, pad_u])
    b_dstp = jnp.concatenate([ii, uu, pad_u])

    xs = jnp.concatenate(
        [jnp.concatenate([user_w, user1_w], 1),
         jnp.zeros((NPS - 50000, F), f32)], 0)
    xu = jnp.concatenate(
        [jnp.concatenate([user_w, user2_w], 1),
         jnp.concatenate([item_w, item2_w], 1),
         jnp.zeros((NPU - 100000, F), f32)], 0)

    ho_s, hi_s = _deg(NRS, ERS, True)(s_srcp, s_dstp)
    (ho_u,) = _deg(NRU, ERU, False)(b_srcp)
    so_s, si_s = _rsqrt(NRS, 2)(ho_s, hi_s)
    (s_u,) = _rsqrt(NRU, 1)(ho_u)
    so_s1 = so_s.reshape(-1)
    si_s1 = si_s.reshape(-1)
    s_u1 = s_u.reshape(-1)

    g0s = _scale(NPS)(xs, so_s1)
    g0u = _scale(NPU)(xu, s_u1)

    bp_s, ln_s = _bin(NPS, ERS, NPS // RNG)(s_srcp, s_dstp)
    bp_u, ln_u = _bin(NPU, ERU, NPU // RNG)(b_srcp, b_dstp)

    h1s, g1s = _hop(NPS, ERS, True)(g0s, bp_s, ln_s, si_s1, so_s1)
    (h2s,) = _hop(NPS, ERS, False)(g1s, bp_s, ln_s, si_s1, so_s1)
    h1u, g1u = _hop(NPU, ERU, True)(g0u, bp_u, ln_u, s_u1, s_u1)
    (h2u,) = _hop(NPU, ERU, False)(g1u, bp_u, ln_u, s_u1, s_u1)

    s64, soc64, u64, a64, itm64, ru64, ri64 = _assemble(
        xs, h1s, h2s, xu, h1u, h2u)

    (users_emb, pos_emb, neg_emb,
     users_soc, pos_soc, neg_soc,
     users_rat, pos_rat, neg_rat) = _gather_many(
        [u64, itm64, itm64, soc64, item1_w, item1_w, ru64, ri64, ri64],
        [users, pos, neg, users, pos, neg, users, pos, neg])

    return (users_emb, pos_emb, neg_emb, s64, a64,
            users_soc, pos_soc, neg_soc,
            users_rat, pos_rat, neg_rat)


# speedup vs baseline: 1.1522x; 1.1522x over previous
import functools

import jax
import jax.numpy as jnp
from jax import lax
from jax.experimental import pallas as pl
from jax.experimental.pallas import tpu as pltpu
from jax.experimental.pallas import tpu_sc as plsc

F = 128
H = 64
RNG = 12544
NPS, ERS, NRS = 50176, 802816, 392
NPU, ERU, NRU = 100352, 1601536, 784
_B, _BPW = 4096, 128
CH = 64

_CP = pltpu.CompilerParams(use_tc_tiling_on_sc=False, needs_layout_passes=False)


def _mesh():
    return plsc.VectorSubcoreMesh(core_axis_name="c", subcore_axis_name="s")


def _wid():
    return lax.axis_index("s") * 2 + lax.axis_index("c")


def _lane(v, j):
    jj = jnp.full((16, 1), j, jnp.int32)
    return lax.gather(
        v, jj,
        lax.GatherDimensionNumbers(offset_dims=(), collapsed_slice_dims=(0,),
                                   start_index_map=(0,)),
        (1,), mode=lax.GatherScatterMode.PROMISE_IN_BOUNDS)


def _z16():
    return jnp.zeros((16,), jnp.float32)


def _ones16():
    return jnp.ones((16,), jnp.float32)


def _deg(n_rows, e_pad, two):
    ept = e_pad // 32
    nch = ept // 128
    nh = 2 if two else 1
    out_type = tuple(jax.ShapeDtypeStruct((32, n_rows, F), jnp.float32)
                     for _ in range(nh))
    scratch = ([pltpu.VMEM((n_rows, F), jnp.float32)] * nh
               + [pltpu.VMEM((128,), jnp.int32)] * nh)

    @functools.partial(pl.kernel, out_type=out_type, mesh=_mesh(),
                       scratch_types=scratch, compiler_params=_CP)
    def k(*refs):
        ins = refs[:nh]
        outs = refs[nh:2 * nh]
        hists = refs[2 * nh:3 * nh]
        idxs = refs[3 * nh:4 * nh]
        w = _wid()

        def zb(i, _):
            for hh in hists:
                for q in range(8):
                    hh[i, pl.ds(q * 16, 16)] = _z16()
            return _
        lax.fori_loop(0, n_rows, zb, None)

        def eb(ch, _):
            off = w * ept + ch * 128
            for t in range(nh):
                pltpu.sync_copy(ins[t].at[pl.ds(off, 128)], idxs[t])
            for t in range(nh):
                for j in range(8):
                    v = idxs[t][pl.ds(j * 16, 16)]
                    plsc.addupdate_scatter(
                        hists[t],
                        [jnp.right_shift(v, 7), jnp.bitwise_and(v, 127)],
                        _ones16())
            return _
        lax.fori_loop(0, nch, eb, None)
        for t in range(nh):
            pltpu.sync_copy(hists[t], outs[t].at[w])

    return k


def _rsqrt(n_rows, nh):
    rb = n_rows // 7

    def body(*refs):
        for t in range(nh):
            deg = jnp.sum(refs[t][...], axis=0)
            refs[nh + t][...] = lax.rsqrt(jnp.maximum(deg, 1.0))

    return pl.pallas_call(
        body, grid=(7,),
        in_specs=[pl.BlockSpec((32, rb, F), lambda i: (0, i, 0))] * nh,
        out_specs=[pl.BlockSpec((rb, F), lambda i: (i, 0))] * nh,
        out_shape=[jax.ShapeDtypeStruct((n_rows, F), jnp.float32)] * nh)


def _scale(n_pad):
    npt = n_pad // 32
    nch = npt // 16
    scratch = [pltpu.VMEM((16, F), jnp.float32), pltpu.VMEM((16,), jnp.float32)]

    @functools.partial(
        pl.kernel, out_type=jax.ShapeDtypeStruct((n_pad, F), jnp.float32),
        mesh=_mesh(), scratch_types=scratch, compiler_params=_CP)
    def k(x_hbm, s_hbm, g_hbm, xr, sv):
        base = _wid() * npt

        def body(ch, _):
            r0 = base + ch * 16
            pltpu.sync_copy(x_hbm.at[pl.ds(r0, 16)], xr)
            pltpu.sync_copy(s_hbm.at[pl.ds(r0, 16)], sv)
            sval = sv[...]
            for j in range(16):
                b = _lane(sval, j)
                for q in range(8):
                    xr[j, pl.ds(q * 16, 16)] = xr[j, pl.ds(q * 16, 16)] * b
            pltpu.sync_copy(xr, g_hbm.at[pl.ds(r0, 16)])
            return _
        lax.fori_loop(0, nch, body, None)

    return k


def _bin(n_pad, e_pad, nb):
    ept2 = e_pad // 32
    nch = ept2 // CH
    cap = ept2 + 128
    i32 = jnp.int32
    out_type = (jax.ShapeDtypeStruct((nb, 32, cap), i32),
                jax.ShapeDtypeStruct((32, 16), i32))
    scratch = [
        pltpu.VMEM((nb, 160), i32),
        pltpu.VMEM((CH,), i32),
        pltpu.VMEM((CH,), i32),
        pltpu.VMEM((16,), i32),
    ]

    @functools.partial(pl.kernel, out_type=out_type, mesh=_mesh(),
                       scratch_types=scratch, compiler_params=_CP)
    def k(srcp, dstp, bp, lens, sbuf, srcv, dstv, lenv):
        w = _wid()
        tb = w * ept2

        def body(ch, carry):
            ptrs = list(carry[:nb])
            wcs = list(carry[nb:])
            off = tb + ch * CH
            pltpu.sync_copy(srcp.at[pl.ds(off, CH)], srcv)
            pltpu.sync_copy(dstp.at[pl.ds(off, CH)], dstv)
            for g in range(CH // 16):
                s16 = srcv[pl.ds(g * 16, 16)]
                d16 = dstv[pl.ds(g * 16, 16)]
                for b in range(nb):
                    lo = b * RNG
                    m = jnp.logical_and(d16 >= lo, d16 < lo + RNG)
                    pk = jnp.bitwise_or(
                        s16, jnp.left_shift(d16 - lo, 17))
                    plsc.store_compressed(sbuf.at[b, pl.ds(ptrs[b], 16)],
                                          pk, mask=m)
                    cnt = lax.reduce_max(
                        plsc.all_reduce_population_count(m), (0,))
                    ptr = ptrs[b] + cnt
                    full = ptr >= 128

                    @pl.when(full)
                    def _(b=b, wc=wcs[b]):
                        pltpu.sync_copy(sbuf.at[b, pl.ds(0, 128)],
                                        bp.at[b, w, pl.ds(wc * 128, 128)])
                        ts = sbuf[b, pl.ds(128, 16)]
                        sbuf[b, pl.ds(0, 16)] = ts
                    ptrs[b] = jnp.where(full, ptr - 128, ptr)
                    wcs[b] = wcs[b] + full.astype(i32)
            return tuple(ptrs) + tuple(wcs)

        z = jnp.zeros((), i32)
        carry = lax.fori_loop(0, nch, body, (z,) * (2 * nb))

        lvec = jnp.zeros((16,), i32)
        for b in range(nb):
            ptr = carry[b]
            wc = carry[nb + b]
            dm16 = jnp.full((16,), (n_pad - 1) | (RNG << 17), i32)
            for kq in range(8):
                @pl.when(ptr + kq * 16 < 128)
                def _(b=b, o=ptr + kq * 16):
                    sbuf[b, pl.ds(o, 16)] = dm16

            @pl.when(ptr > 0)
            def _(b=b, wc=wc):
                pltpu.sync_copy(sbuf.at[b, pl.ds(0, 128)],
                                bp.at[b, w, pl.ds(wc * 128, 128)])
            nchunks = wc + (ptr > 0).astype(i32)
            lvec = lvec + nchunks * jnp.where(
                lax.iota(i32, 16) == b, 1, 0)
        lenv[...] = lvec
        pltpu.sync_copy(lenv, lens.at[w])

    return k


def _hop(n_pad, e_pad, emit_g):
    nb = n_pad // RNG
    npp = nb // 2
    cap = e_pad // 32 + 128
    rpt = RNG // 16
    nfl = rpt // 16
    out_type = tuple(jax.ShapeDtypeStruct((n_pad, F), jnp.float32)
                     for _ in range(2 if emit_g else 1))
    scratch = [
        pltpu.VMEM_SHARED((RNG + 8, F), jnp.float32),
        pltpu.VMEM((CH,), jnp.int32),
        pltpu.VMEM((CH,), jnp.int32),
        pltpu.VMEM((CH,), jnp.int32),
        pltpu.VMEM((CH,), jnp.int32),
        pltpu.VMEM((CH,), jnp.int32),
        pltpu.VMEM((CH, F), jnp.float32),
        pltpu.VMEM((CH, F), jnp.float32),
        pltpu.VMEM((16, F), jnp.float32),
        pltpu.VMEM((16, F), jnp.float32),
        pltpu.VMEM((16, F), jnp.float32),
        pltpu.VMEM((16, F), jnp.float32),
        pltpu.VMEM((16,), jnp.float32),
        pltpu.VMEM((16,), jnp.float32),
        pltpu.VMEM((16,), jnp.int32),
        pltpu.VMEM((16,), jnp.int32),
        pltpu.SemaphoreType.DMA,
        pltpu.SemaphoreType.DMA,
    ]

    @functools.partial(pl.kernel, out_type=out_type, mesh=_mesh(),
                       scratch_types=scratch, compiler_params=_CP)
    def k(*refs):
        g_hbm, bp_hbm, lens_hbm, si_hbm, so_hbm = refs[:5]
        if emit_g:
            h_hbm, gn_hbm = refs[5:7]
            sc = refs[7:]
        else:
            h_hbm = refs[5]
            gn_hbm = None
            sc = refs[6:]
        (acc, srcv0, srcv1, dstv, dstl0, dstl1, rows0, rows1,
         tv, hv, gv, z16, siv, sov, lenv0, lenv1, sem0, sem1) = sc
        cid = lax.axis_index("c")
        tid = lax.axis_index("s")

        for r in range(16):
            for q in range(8):
                z16[r, pl.ds(q * 16, 16)] = _z16()

        def zinit(q, _):
            pltpu.sync_copy(z16, acc.at[pl.ds(tid * rpt + q * 16, 16)])
            return _
        lax.fori_loop(0, nfl, zinit, None)

        @pl.when(tid == 0)
        def _():
            pltpu.sync_copy(z16.at[pl.ds(0, 8)], acc.at[pl.ds(RNG, 8)])
        plsc.subcore_barrier()

        i32 = jnp.int32
        pltpu.sync_copy(lens_hbm.at[2 * tid], lenv0)
        pltpu.sync_copy(lens_hbm.at[2 * tid + 1], lenv1)
        lv0 = lenv0[...]
        lv1 = lenv1[...]

        for p in range(npp):
            bkt = cid * npp + p
            base = bkt * RNG
            oh0 = jnp.where(lax.iota(i32, 16) == p, 1, 0)
            oh1 = jnp.where(lax.iota(i32, 16) == npp + p, 1, 0)
            oh = jnp.where(cid == 0, oh0, oh1)
            n0 = lax.reduce_max(lv0 * oh, (0,)) * (128 // CH)
            n1 = lax.reduce_max(lv1 * oh, (0,)) * (128 // CH)
            ntot = n0 + n1

            def prep(c, srcv, dstl, sem):
                in1 = (c >= n0).astype(i32)
                w = 2 * tid + in1
                o = (c - in1 * n0) * CH
                pltpu.sync_copy(bp_hbm.at[bkt, w, pl.ds(o, CH)], dstv)
                for j in range(CH // 16):
                    pk = dstv[pl.ds(j * 16, 16)]
                    srcv[pl.ds(j * 16, 16)] = jnp.bitwise_and(pk, 0x1FFFF)
                    dstl[pl.ds(j * 16, 16)] = jnp.right_shift(pk, 17)
                rows = rows0 if srcv is srcv0 else rows1
                return pltpu.async_copy(g_hbm.at[srcv], rows, sem)

            @pl.when(ntot > 0)
            def _():
                prep(jnp.zeros((), i32), srcv0, dstl0, sem0)

            def scan_body(i2, _):
                c1 = 2 * i2 + 1

                @pl.when(c1 < ntot)
                def _():
                    prep(c1, srcv1, dstl1, sem1)
                pltpu.make_async_copy(g_hbm.at[srcv0], rows0, sem0).wait()
                pltpu.sync_copy(rows0, acc.at[dstl0], add=True)

                @pl.when(2 * i2 + 2 < ntot)
                def _():
                    prep(2 * i2 + 2, srcv0, dstl0, sem0)

                @pl.when(c1 < ntot)
                def _():
                    pltpu.make_async_copy(g_hbm.at[srcv1], rows1, sem1).wait()
                    pltpu.sync_copy(rows1, acc.at[dstl1], add=True)
                return _
            lax.fori_loop(0, (ntot + 1) // 2, scan_body, None)
            plsc.subcore_barrier()

            def fl(q, _):
                r0 = tid * rpt + q * 16
                n0 = base + r0
                pltpu.sync_copy(acc.at[pl.ds(r0, 16)], tv)
                pltpu.sync_copy(z16, acc.at[pl.ds(r0, 16)])
                pltpu.sync_copy(si_hbm.at[pl.ds(n0, 16)], siv)
                if emit_g:
                    pltpu.sync_copy(so_hbm.at[pl.ds(n0, 16)], sov)
                sival = siv[...]
                soval = sov[...] if emit_g else None
                for j in range(16):
                    a = _lane(sival, j)
                    if emit_g:
                        b = _lane(soval, j) * a
                    for q2 in range(8):
                        t = tv[j, pl.ds(q2 * 16, 16)]
                        hv[j, pl.ds(q2 * 16, 16)] = t * a
                        if emit_g:
                            gv[j, pl.ds(q2 * 16, 16)] = t * b
                pltpu.sync_copy(hv, h_hbm.at[pl.ds(n0, 16)])
                if emit_g:
                    pltpu.sync_copy(gv, gn_hbm.at[pl.ds(n0, 16)])
                return _
            lax.fori_loop(0, nfl, fl, None)

            @pl.when(tid == 0)
            def _():
                pltpu.sync_copy(z16.at[pl.ds(0, 8)], acc.at[pl.ds(RNG, 8)])
            plsc.subcore_barrier()

    return k


def _assemble(xs, h1s, h2s, xu, h1u, h2u):
    rb = 2000
    grid = 50000 // rb

    def body(axs, ah1s, ah2s, au0, au1, au2, ai0, ai1, ai2,
             s64, soc, u64, a64, itm, ru, ri):
        fs = (axs[...] + ah1s[...] + ah2s[...]) * (1.0 / 3.0)
        fu = (au0[...] + au1[...] + au2[...]) * (1.0 / 3.0)
        fi = (ai0[...] + ai1[...] + ai2[...]) * (1.0 / 3.0)
        s64[...] = fs[:, :H]
        soc[...] = fs[:, H:]
        a64[...] = fu[:, :H]
        ru[...] = fu[:, H:]
        itm[...] = fi[:, :H]
        ri[...] = fi[:, H:]
        u64[...] = 0.5 * fs[:, :H] + 0.5 * fu[:, :H]

    uspec = pl.BlockSpec((rb, F), lambda i: (i, 0))
    ispec = pl.BlockSpec((rb, F), lambda i: (grid + i, 0))
    ospec = pl.BlockSpec((rb, H), lambda i: (i, 0))
    return pl.pallas_call(
        body, grid=(grid,),
        in_specs=[uspec] * 6 + [ispec] * 3,
        out_specs=[ospec] * 7,
        out_shape=[jax.ShapeDtypeStruct((50000, H), jnp.float32)] * 7,
        compiler_params=pltpu.CompilerParams(
            vmem_limit_bytes=100 * 1024 * 1024),
    )(xs, h1s, h2s, xu, h1u, h2u, xu, h1u, h2u)


def _gather_many(tables, idxs):
    nt = len(tables)
    out_type = tuple(
        jax.ShapeDtypeStruct((idxs[t].shape[0], tables[t].shape[1]),
                             jnp.float32) for t in range(nt))
    scratch = [
        pltpu.VMEM((_BPW,), jnp.int32),
        pltpu.VMEM((_BPW, H), jnp.float32),
        pltpu.SemaphoreType.DMA,
    ]

    @functools.partial(pl.kernel, out_type=out_type, mesh=_mesh(),
                       scratch_types=scratch, compiler_params=_CP)
    def k(*refs):
        tabs = refs[:nt]
        idxr = refs[nt:2 * nt]
        outs = refs[2 * nt:3 * nt]
        idx_v, rows_v, sem = refs[3 * nt:]
        base = _wid() * _BPW
        for t in range(nt):
            pltpu.sync_copy(idxr[t].at[pl.ds(base, _BPW)], idx_v)
            pltpu.async_copy(tabs[t].at[idx_v], rows_v, sem).wait()
            pltpu.sync_copy(rows_v, outs[t].at[pl.ds(base, _BPW)])

    return k(*tables, *idxs)


def kernel(users, pos, neg, social_edge_index, ui_edge_index,
           user_w, item_w, user1_w, item1_w, user2_w, item2_w):
    i32 = jnp.int32
    f32 = jnp.float32
    users = users.astype(i32)
    pos = pos.astype(i32)
    neg = neg.astype(i32)

    s_src = social_edge_index[0].astype(i32)
    s_dst = social_edge_index[1].astype(i32)
    pad_s = jnp.full((ERS - s_src.shape[0],), NPS - 1, i32)
    s_srcp = jnp.concatenate([s_src, pad_s])
    s_dstp = jnp.concatenate([s_dst, pad_s])

    uu = ui_edge_index[0].astype(i32)
    ii = ui_edge_index[1].astype(i32) + 50000
    pad_u = jnp.full((ERU - 2 * uu.shape[0],), NPU - 1, i32)
    b_srcp = jnp.concatenate([uu, ii, pad_u])
    b_dstp = jnp.concatenate([ii, uu, pad_u])

    xs = jnp.concatenate(
        [jnp.concatenate([user_w, user1_w], 1),
         jnp.zeros((NPS - 50000, F), f32)], 0)
    xu = jnp.concatenate(
        [jnp.concatenate([user_w, user2_w], 1),
         jnp.concatenate([item_w, item2_w], 1),
         jnp.zeros((NPU - 100000, F), f32)], 0)

    ho_s, hi_s = _deg(NRS, ERS, True)(s_srcp, s_dstp)
    (ho_u,) = _deg(NRU, ERU, False)(b_srcp)
    so_s, si_s = _rsqrt(NRS, 2)(ho_s, hi_s)
    (s_u,) = _rsqrt(NRU, 1)(ho_u)
    so_s1 = so_s.reshape(-1)
    si_s1 = si_s.reshape(-1)
    s_u1 = s_u.reshape(-1)

    g0s = _scale(NPS)(xs, so_s1)
    g0u = _scale(NPU)(xu, s_u1)

    bp_s, ln_s = _bin(NPS, ERS, NPS // RNG)(s_srcp, s_dstp)
    bp_u, ln_u = _bin(NPU, ERU, NPU // RNG)(b_srcp, b_dstp)

    h1s, g1s = _hop(NPS, ERS, True)(g0s, bp_s, ln_s, si_s1, so_s1)
    (h2s,) = _hop(NPS, ERS, False)(g1s, bp_s, ln_s, si_s1, so_s1)
    h1u, g1u = _hop(NPU, ERU, True)(g0u, bp_u, ln_u, s_u1, s_u1)
    (h2u,) = _hop(NPU, ERU, False)(g1u, bp_u, ln_u, s_u1, s_u1)

    s64, soc64, u64, a64, itm64, ru64, ri64 = _assemble(
        xs, h1s, h2s, xu, h1u, h2u)

    (users_emb, pos_emb, neg_emb,
     users_soc, pos_soc, neg_soc,
     users_rat, pos_rat, neg_rat) = _gather_many(
        [u64, itm64, itm64, soc64, item1_w, item1_w, ru64, ri64, ri64],
        [users, pos, neg, users, pos, neg, users, pos, neg])

    return (users_emb, pos_emb, neg_emb, s64, a64,
            users_soc, pos_soc, neg_soc,
            users_rat, pos_rat, neg_rat)
